# baseline (speedup 1.0000x reference)
import jax
import jax.numpy as jnp
from jax.experimental import pallas as pl

INPUT_DIM = 2048
NUM_EXPERTS = 64
BLOCK_T = 2048


def _router_body(x_ref, wt_ref, disp_ref, probs_ref, sel_ref, w_ref):
    logits = jnp.dot(x_ref[...], wt_ref[...], preferred_element_type=jnp.float32)
    m = jnp.max(logits, axis=1, keepdims=True)
    e = jnp.exp(logits - m)
    probs = e / jnp.sum(e, axis=1, keepdims=True)
    probs_ref[...] = probs

    eidf = jax.lax.broadcasted_iota(jnp.int32, probs.shape, 1).astype(jnp.float32)
    riota = 63.0 - eidf
    p1 = jnp.max(probs, axis=1, keepdims=True)
    i1f = 63.0 - jnp.max(jnp.where(probs == p1, riota, -1.0), axis=1, keepdims=True)
    masked = jnp.where(eidf == i1f, -1.0, probs)
    p2 = jnp.max(masked, axis=1, keepdims=True)
    i2f = 63.0 - jnp.max(jnp.where(masked == p2, riota, -1.0), axis=1, keepdims=True)

    denom = p1 + p2
    w1 = p1 / denom
    w2 = p2 / denom
    disp_ref[...] = jnp.where(
        eidf == i1f, w1, jnp.where(eidf == i2f, w2, jnp.zeros_like(probs))
    )
    sel_ref[...] = jnp.concatenate([i1f, i2f], axis=1).astype(jnp.int32).T
    w_ref[...] = jnp.concatenate([w1, w2], axis=1).T


@jax.jit
def kernel(x, W):
    B, S, D = x.shape
    T = B * S
    x2 = x.reshape(T, D)
    wt = W.T
    disp, probs, sel_t, wts_t = pl.pallas_call(
        _router_body,
        grid=(T // BLOCK_T,),
        in_specs=[
            pl.BlockSpec((BLOCK_T, D), lambda i: (i, 0)),
            pl.BlockSpec((D, NUM_EXPERTS), lambda i: (0, 0)),
        ],
        out_specs=[
            pl.BlockSpec((BLOCK_T, NUM_EXPERTS), lambda i: (i, 0)),
            pl.BlockSpec((BLOCK_T, NUM_EXPERTS), lambda i: (i, 0)),
            pl.BlockSpec((2, BLOCK_T), lambda i: (0, i)),
            pl.BlockSpec((2, BLOCK_T), lambda i: (0, i)),
        ],
        out_shape=[
            jax.ShapeDtypeStruct((T, NUM_EXPERTS), jnp.float32),
            jax.ShapeDtypeStruct((T, NUM_EXPERTS), jnp.float32),
            jax.ShapeDtypeStruct((2, T), jnp.int32),
            jax.ShapeDtypeStruct((2, T), jnp.float32),
        ],
    )(x2, wt)
    return (
        disp.reshape(B, S, NUM_EXPERTS),
        probs.reshape(B, S, NUM_EXPERTS),
        sel_t.T.reshape(B, S, 2),
        wts_t.T.reshape(B, S, 2),
    )
